# baseline (speedup 1.0000x reference)
import functools

import jax
import jax.numpy as jnp
from jax import lax
from jax.experimental import pallas as pl
from jax.experimental.pallas import tpu as pltpu
from jax.experimental.pallas import tpu_sc as plsc

_B, _M, _D, _K = 1024, 64, 32, 512
_MG = 8


_BB = 256


def _assign_body(zt_ref, cb_ref, idx_ref, flat_ref, loss_ref):
    mg = pl.program_id(0)
    bb = pl.program_id(1)
    part = jnp.float32(0.0)
    for j in range(_MG):
        m = mg * _MG + j
        zm = zt_ref[j]
        cbm = cb_ref[j]
        cn = jnp.sum(cbm * cbm, axis=1, keepdims=True)
        scores = lax.dot_general(
            cbm * -2.0, zm, (((1,), (1,)), ((), ())),
            preferred_element_type=jnp.float32,
            precision=lax.Precision.HIGHEST,
        )
        dist = cn + scores
        minv = jnp.min(dist, axis=0, keepdims=True)
        iota_k = lax.broadcasted_iota(jnp.int32, (_K, _BB), 0)
        idx = jnp.min(jnp.where(dist == minv, iota_k, _K), axis=0)
        idx_ref[j, 0, :] = idx
        flat_ref[j, 0, :] = idx + m * _K
        part = part + (jnp.sum(minv) + jnp.sum(zm * zm))

    @pl.when(jnp.logical_and(mg == 0, bb == 0))
    def _():
        loss_ref[:, :] = jnp.zeros((1, 1), jnp.float32)

    loss_ref[:, :] = loss_ref[:, :] + part


_assign_call = pl.pallas_call(
    _assign_body,
    grid=(_M // _MG, _B // _BB),
    in_specs=[
        pl.BlockSpec((_MG, _BB, _D), lambda mg, bb: (mg, bb, 0)),
        pl.BlockSpec((_MG, _K, _D), lambda mg, bb: (mg, 0, 0)),
    ],
    out_specs=[
        pl.BlockSpec((_MG, 1, _BB), lambda mg, bb: (mg, 0, bb)),
        pl.BlockSpec((_MG, 1, _BB), lambda mg, bb: (mg, 0, bb)),
        pl.BlockSpec((1, 1), lambda mg, bb: (0, 0)),
    ],
    out_shape=[
        jax.ShapeDtypeStruct((_M, 1, _B), jnp.int32),
        jax.ShapeDtypeStruct((_M, 1, _B), jnp.int32),
        jax.ShapeDtypeStruct((1, 1), jnp.float32),
    ],
)


@functools.lru_cache(maxsize=1)
def _make_sc_gather():
    nc, ns = 2, 16
    nw = nc * ns
    n = _B * _M
    bpw = n // nw
    ch = 64
    nch = bpw // ch
    mesh = plsc.VectorSubcoreMesh(
        core_axis_name="c", subcore_axis_name="s",
        num_cores=nc, num_subcores=ns,
    )

    @functools.partial(
        pl.kernel,
        mesh=mesh,
        compiler_params=pltpu.CompilerParams(use_tc_tiling_on_sc=False),
        out_type=jax.ShapeDtypeStruct((_B, _M, _D), jnp.float32),
        scratch_types=[
            pltpu.VMEM((nch, ch), jnp.int32),
            pltpu.VMEM((nch, ch, _D), jnp.float32),
            pltpu.SemaphoreType.DMA,
        ],
    )
    def gather(table_hbm, idx_hbm, out_hbm, idx_v, rows_v, sem):
        wid = lax.axis_index("s") * nc + lax.axis_index("c")
        pltpu.sync_copy(idx_hbm.at[wid], idx_v)
        copies = [
            pltpu.async_copy(table_hbm.at[idx_v.at[j]], rows_v.at[j], sem)
            for j in range(nch)
        ]
        for c in copies:
            c.wait()
        pltpu.sync_copy(rows_v, out_hbm.at[pl.ds(wid * (nch), nch)])

    return gather, nw, nch, ch


def kernel(z, codebooks):
    sc_gather, nw, nch, ch = _make_sc_gather()
    z_t = jnp.transpose(z, (1, 0, 2))
    idx_mb, flat_mb, loss = _assign_call(z_t, codebooks)
    idx = idx_mb.reshape(_M, _B).T
    flat = flat_mb.reshape(_M, _B).T.reshape(nw, nch, ch)
    table = codebooks.reshape(_M * _K, _D)
    quantized = sc_gather(table, flat)
    q_loss = (loss[0, 0] * (1.25 / (_B * _M * _D))).astype(jnp.float32)
    return quantized, idx, q_loss


# speedup vs baseline: 1.0843x; 1.0843x over previous
import functools

import jax
import jax.numpy as jnp
from jax import lax
from jax.experimental import pallas as pl
from jax.experimental.pallas import tpu as pltpu
from jax.experimental.pallas import tpu_sc as plsc

_B, _M, _D, _K = 1024, 64, 32, 512
_MG = 8


def _assign_body(zt_ref, cb_ref, idx_ref, flat_ref, loss_ref):
    g = pl.program_id(0)
    part = jnp.float32(0.0)
    for j in range(_MG):
        m = g * _MG + j
        zm = zt_ref[j]
        cbm = cb_ref[j]
        cn = jnp.sum(cbm * cbm, axis=1, keepdims=True)
        scores = lax.dot_general(
            cbm * -2.0, zm, (((1,), (1,)), ((), ())),
            preferred_element_type=jnp.float32,
            precision=lax.Precision.HIGHEST,
        )
        dist = cn + scores
        minv = jnp.min(dist, axis=0, keepdims=True)
        iota_k = lax.broadcasted_iota(jnp.int32, (_K, _B), 0)
        idx = jnp.min(jnp.where(dist == minv, iota_k, _K), axis=0)
        idx_ref[j, 0, :] = idx
        flat_ref[j, 0, :] = idx + m * _K
        part = part + (jnp.sum(minv) + jnp.sum(zm * zm))

    @pl.when(g == 0)
    def _():
        loss_ref[:, :] = jnp.zeros((1, 1), jnp.float32)

    loss_ref[:, :] = loss_ref[:, :] + part


_assign_call = pl.pallas_call(
    _assign_body,
    grid=(_M // _MG,),
    in_specs=[
        pl.BlockSpec((_MG, _B, _D), lambda g: (g, 0, 0)),
        pl.BlockSpec((_MG, _K, _D), lambda g: (g, 0, 0)),
    ],
    out_specs=[
        pl.BlockSpec((_MG, 1, _B), lambda g: (g, 0, 0)),
        pl.BlockSpec((_MG, 1, _B), lambda g: (g, 0, 0)),
        pl.BlockSpec((1, 1), lambda g: (0, 0)),
    ],
    out_shape=[
        jax.ShapeDtypeStruct((_M, 1, _B), jnp.int32),
        jax.ShapeDtypeStruct((_M, 1, _B), jnp.int32),
        jax.ShapeDtypeStruct((1, 1), jnp.float32),
    ],
)


@functools.lru_cache(maxsize=1)
def _make_sc_gather():
    nc, ns = 2, 16
    nw = nc * ns
    n = _B * _M
    bpw = n // nw
    ch = 64
    nch = bpw // ch
    mesh = plsc.VectorSubcoreMesh(
        core_axis_name="c", subcore_axis_name="s",
        num_cores=nc, num_subcores=ns,
    )

    @functools.partial(
        pl.kernel,
        mesh=mesh,
        compiler_params=pltpu.CompilerParams(use_tc_tiling_on_sc=False),
        out_type=jax.ShapeDtypeStruct((_B, _M, _D), jnp.float32),
        scratch_types=[
            pltpu.VMEM((nch, ch), jnp.int32),
            pltpu.VMEM((nch, ch, _D), jnp.float32),
            pltpu.SemaphoreType.DMA,
        ],
    )
    def gather(table_hbm, idx_hbm, out_hbm, idx_v, rows_v, sem):
        wid = lax.axis_index("s") * nc + lax.axis_index("c")
        pltpu.sync_copy(idx_hbm.at[wid], idx_v)
        copies = [
            pltpu.async_copy(table_hbm.at[idx_v.at[j]], rows_v.at[j], sem)
            for j in range(nch)
        ]
        for c in copies:
            c.wait()
        pltpu.sync_copy(rows_v, out_hbm.at[pl.ds(wid * (nch), nch)])

    return gather, nw, nch, ch


def kernel(z, codebooks):
    sc_gather, nw, nch, ch = _make_sc_gather()
    z_t = jnp.transpose(z, (1, 0, 2))
    idx_mb, flat_mb, loss = _assign_call(z_t, codebooks)
    idx = idx_mb.reshape(_M, _B).T
    flat = flat_mb.reshape(_M, _B).T.reshape(nw, nch, ch)
    table = codebooks.reshape(_M * _K, _D)
    quantized = sc_gather(table, flat)
    q_loss = (loss[0, 0] * (1.25 / (_B * _M * _D))).astype(jnp.float32)
    return quantized, idx, q_loss
